# baseline (speedup 1.0000x reference)
import functools

import jax
import jax.numpy as jnp
from jax import lax
from jax.experimental import pallas as pl
from jax.experimental.pallas import tpu as pltpu
from jax.experimental.pallas import tpu_sc as plsc

B, C, H, W = 16, 3, 512, 512
BINS = 64
ROWS = B * C

NC, NS, L = 2, 16, 16
CH_ROWS = 64
CHUNK = CH_ROWS * W
CHUNKS_PER_CH = H // CH_ROWS
VECS_PER_ROW = W // L
TBL = C * BINS


def _histograms_sc(fake_images, real_images):
  mesh = plsc.VectorSubcoreMesh(
      core_axis_name="c", subcore_axis_name="s", num_cores=NC,
      num_subcores=NS)

  @functools.partial(
      pl.kernel,
      out_type=jax.ShapeDtypeStruct((NC * NS, TBL), jnp.float32),
      mesh=mesh,
      compiler_params=pltpu.CompilerParams(
          needs_layout_passes=False, use_tc_tiling_on_sc=True),
      scratch_types=[
          pltpu.VMEM((CH_ROWS, W), jnp.float32),
          pltpu.VMEM((CH_ROWS, W), jnp.float32),
          pltpu.VMEM((L * TBL,), jnp.float32),
          pltpu.VMEM((TBL,), jnp.float32),
          pltpu.SemaphoreType.DMA,
          pltpu.SemaphoreType.DMA,
      ],
  )
  def hist_kernel(fake_hbm, real_hbm, out_hbm, buf0, buf1, table, acc,
                  sem0, sem1):
    c = lax.axis_index("c")
    s = lax.axis_index("s")
    bufs = (buf0, buf1)
    sems = (sem0, sem1)
    lane_base = jnp.arange(L, dtype=jnp.int32) * TBL
    ones = jnp.ones((L,), jnp.float32)
    zeros = jnp.zeros((L,), jnp.float32)
    nchunks = C * CHUNKS_PER_CH

    @plsc.parallel_loop(0, TBL, unroll=8)
    def _(j):
      table[pl.ds(j * L, L)] = zeros

    def run(img_hbm):

      def start(k, nb):
        ch = lax.shift_right_logical(k, 3)
        blk = lax.bitwise_and(k, CHUNKS_PER_CH - 1)
        pltpu.async_copy(
            img_hbm.at[s, ch, pl.ds(blk * CH_ROWS, CH_ROWS), :],
            bufs[nb], sems[nb])

      def wait(nb):
        pltpu.make_async_copy(
            img_hbm.at[0, 0, pl.ds(0, CH_ROWS), :], bufs[nb],
            sems[nb]).wait()

      def process(k, nb):
        cvec = lane_base + lax.shift_right_logical(k, 3) * BINS

        @plsc.parallel_loop(0, CH_ROWS)
        def _(r):
          for col in range(VECS_PER_ROW):
            x = bufs[nb][r, pl.ds(col * L, L)]
            idx = (x * jnp.float32(BINS)).astype(jnp.int32)
            idx = plsc.bitcast(
                jnp.minimum(plsc.bitcast(idx, jnp.uint32),
                            jnp.uint32(BINS - 1)), jnp.int32)
            plsc.addupdate_scatter(table, [idx + cvec], ones)

      start(jnp.int32(0), 0)

      def outer(k2, carry):
        a = k2 * 2
        start(a + 1, 1)
        wait(0)
        process(a, 0)

        @pl.when(k2 < nchunks // 2 - 1)
        def _():
          start(a + 2, 0)

        wait(1)
        process(a + 1, 1)
        return carry

      lax.fori_loop(0, nchunks // 2, outer, 0)

    @pl.when(c == 0)
    def _():
      run(fake_hbm)

    @pl.when(c == 1)
    def _():
      run(real_hbm)

    for j in range(TBL // L):
      v = table[pl.ds(j * L, L)]
      for l in range(1, L):
        v = v + table[pl.ds(l * TBL + j * L, L)]
      acc[pl.ds(j * L, L)] = v

    pltpu.sync_copy(acc, out_hbm.at[c * NS + s])

  return hist_kernel(fake_images, real_images)


def _loss_body(h_ref, o_ref):
  h = h_ref[...]
  ssum = jnp.clip(jnp.sum(h, axis=1, keepdims=True), 1e-8, None)
  n = h / ssum
  d = jnp.abs(n[:ROWS] - n[ROWS:])
  o_ref[0, 0] = jnp.sum(d) / jnp.float32(ROWS * BINS)


def kernel(fake_images, real_images):
  hists = _histograms_sc(fake_images, real_images).reshape(NC * ROWS, BINS)
  loss = pl.pallas_call(
      _loss_body,
      out_shape=jax.ShapeDtypeStruct((1, 1), jnp.float32),
      out_specs=pl.BlockSpec(memory_space=pltpu.SMEM),
  )(hists)
  return loss[0, 0]


# speedup vs baseline: 1.0439x; 1.0439x over previous
import functools

import jax
import jax.numpy as jnp
from jax import lax
from jax.experimental import pallas as pl
from jax.experimental.pallas import tpu as pltpu
from jax.experimental.pallas import tpu_sc as plsc

B, C, H, W = 16, 3, 512, 512
BINS = 64
ROWS = B * C

NC, NS, L = 2, 16, 16
CH_ROWS = 64
CHUNK = CH_ROWS * W
CHUNKS_PER_CH = H // CH_ROWS
VECS_PER_ROW = W // L
TBL = C * BINS


def _histograms_sc(fake_images, real_images):
  mesh = plsc.VectorSubcoreMesh(
      core_axis_name="c", subcore_axis_name="s", num_cores=NC,
      num_subcores=NS)

  @functools.partial(
      pl.kernel,
      out_type=jax.ShapeDtypeStruct((NC * NS, TBL), jnp.float32),
      mesh=mesh,
      compiler_params=pltpu.CompilerParams(
          needs_layout_passes=False, use_tc_tiling_on_sc=True),
      scratch_types=[
          pltpu.VMEM((CH_ROWS, W), jnp.float32),
          pltpu.VMEM((CH_ROWS, W), jnp.float32),
          pltpu.VMEM((L * TBL,), jnp.float32),
          pltpu.VMEM((TBL,), jnp.float32),
          pltpu.SemaphoreType.DMA,
          pltpu.SemaphoreType.DMA,
      ],
  )
  def hist_kernel(fake_hbm, real_hbm, out_hbm, buf0, buf1, table, acc,
                  sem0, sem1):
    c = lax.axis_index("c")
    s = lax.axis_index("s")
    bufs = (buf0, buf1)
    sems = (sem0, sem1)
    lane_iota = jnp.arange(L, dtype=jnp.int32)
    ones = jnp.ones((L,), jnp.float32)
    zeros = jnp.zeros((L,), jnp.float32)
    nchunks = C * CHUNKS_PER_CH

    @plsc.parallel_loop(0, TBL, unroll=8)
    def _(j):
      table[pl.ds(j * L, L)] = zeros

    def run(img_hbm):

      def start(k, nb):
        ch = lax.shift_right_logical(k, 3)
        blk = lax.bitwise_and(k, CHUNKS_PER_CH - 1)
        pltpu.async_copy(
            img_hbm.at[s, ch, pl.ds(blk * CH_ROWS, CH_ROWS), :],
            bufs[nb], sems[nb])

      def wait(nb):
        pltpu.make_async_copy(
            img_hbm.at[0, 0, pl.ds(0, CH_ROWS), :], bufs[nb],
            sems[nb]).wait()

      def process(k, nb):
        cvec = lane_iota + lax.shift_right_logical(k, 3) * (BINS * L)

        @plsc.parallel_loop(0, CH_ROWS)
        def _(r):
          for col in range(VECS_PER_ROW):
            x = bufs[nb][r, pl.ds(col * L, L)]
            idx = (x * jnp.float32(BINS)).astype(jnp.int32)
            idx = plsc.bitcast(
                jnp.minimum(plsc.bitcast(idx, jnp.uint32),
                            jnp.uint32(BINS - 1)), jnp.int32)
            addr = lax.shift_left(idx, 4) + cvec
            plsc.addupdate_scatter(table, [addr], ones)

      start(jnp.int32(0), 0)

      def outer(k2, carry):
        a = k2 * 2
        start(a + 1, 1)
        wait(0)
        process(a, 0)

        @pl.when(k2 < nchunks // 2 - 1)
        def _():
          start(a + 2, 0)

        wait(1)
        process(a + 1, 1)
        return carry

      lax.fori_loop(0, nchunks // 2, outer, 0)

    @pl.when(c == 0)
    def _():
      run(fake_hbm)

    @pl.when(c == 1)
    def _():
      run(real_hbm)

    for j in range(TBL // L):
      v = zeros
      for d in range(L):
        diag = (jnp.arange(L, dtype=jnp.int32) * L + j * L * L +
                (jnp.arange(L, dtype=jnp.int32) + d) % L)
        v = v + plsc.load_gather(table, [diag])
      acc[pl.ds(j * L, L)] = v

    pltpu.sync_copy(acc, out_hbm.at[c * NS + s])

  return hist_kernel(fake_images, real_images)


def _loss_body(h_ref, o_ref):
  h = h_ref[...]
  ssum = jnp.clip(jnp.sum(h, axis=1, keepdims=True), 1e-8, None)
  n = h / ssum
  d = jnp.abs(n[:ROWS] - n[ROWS:])
  o_ref[0, 0] = jnp.sum(d) / jnp.float32(ROWS * BINS)


def kernel(fake_images, real_images):
  hists = _histograms_sc(fake_images, real_images).reshape(NC * ROWS, BINS)
  loss = pl.pallas_call(
      _loss_body,
      out_shape=jax.ShapeDtypeStruct((1, 1), jnp.float32),
      out_specs=pl.BlockSpec(memory_space=pltpu.SMEM),
  )(hists)
  return loss[0, 0]


# speedup vs baseline: 1.1690x; 1.1198x over previous
import functools

import jax
import jax.numpy as jnp
from jax import lax
from jax.experimental import pallas as pl
from jax.experimental.pallas import tpu as pltpu
from jax.experimental.pallas import tpu_sc as plsc

B, C, H, W = 16, 3, 512, 512
BINS = 64
ROWS = B * C

NC, NS, L = 2, 16, 16
CH_ROWS = 64
CHUNK = CH_ROWS * W
CHUNKS_PER_CH = H // CH_ROWS
CH_SHIFT = 3
NBUF = 3
VECS_PER_ROW = W // L
TBL = C * BINS


def _histograms_sc(fake_images, real_images):
  mesh = plsc.VectorSubcoreMesh(
      core_axis_name="c", subcore_axis_name="s", num_cores=NC,
      num_subcores=NS)

  @functools.partial(
      pl.kernel,
      out_type=jax.ShapeDtypeStruct((NC * NS, TBL), jnp.float32),
      mesh=mesh,
      compiler_params=pltpu.CompilerParams(
          needs_layout_passes=False, use_tc_tiling_on_sc=True),
      scratch_types=[
          [pltpu.VMEM((CH_ROWS, W), jnp.float32)] * NBUF,
          pltpu.VMEM((L * TBL,), jnp.float32),
          pltpu.VMEM((TBL,), jnp.float32),
          [pltpu.SemaphoreType.DMA] * NBUF,
      ],
  )
  def hist_kernel(fake_hbm, real_hbm, out_hbm, bufs, table, acc, sems):
    c = lax.axis_index("c")
    s = lax.axis_index("s")
    lane_iota = jnp.arange(L, dtype=jnp.int32)
    ones = jnp.ones((L,), jnp.float32)
    zeros = jnp.zeros((L,), jnp.float32)
    nchunks = C * CHUNKS_PER_CH

    @plsc.parallel_loop(0, TBL, unroll=8)
    def _(j):
      table[pl.ds(j * L, L)] = zeros

    def run(img_hbm):

      def start(k, nb):
        ch = lax.shift_right_logical(k, CH_SHIFT)
        blk = lax.bitwise_and(k, CHUNKS_PER_CH - 1)
        pltpu.async_copy(
            img_hbm.at[s, ch, pl.ds(blk * CH_ROWS, CH_ROWS), :],
            bufs[nb], sems[nb])

      def wait(nb):
        pltpu.make_async_copy(
            img_hbm.at[0, 0, pl.ds(0, CH_ROWS), :], bufs[nb],
            sems[nb]).wait()

      def process(k, nb):
        cvec = lane_iota + lax.shift_right_logical(k, CH_SHIFT) * (BINS * L)

        @plsc.parallel_loop(0, CH_ROWS)
        def _(r):

          @plsc.parallel_loop(0, VECS_PER_ROW, unroll=8)
          def _(col):
            x = bufs[nb][r, pl.ds(col * L, L)]
            idx = (x * jnp.float32(BINS)).astype(jnp.int32)
            idx = plsc.bitcast(
                jnp.minimum(plsc.bitcast(idx, jnp.uint32),
                            jnp.uint32(BINS - 1)), jnp.int32)
            addr = lax.shift_left(idx, 4) + cvec
            plsc.addupdate_scatter(table, [addr], ones)

      for i in range(NBUF - 1):
        start(jnp.int32(i), i)

      def outer(kq, carry):
        a = kq * NBUF
        for ph in range(NBUF):

          @pl.when(a + ph + NBUF - 1 < nchunks)
          def _(ph=ph):
            start(a + ph + NBUF - 1, (ph + NBUF - 1) % NBUF)

          wait(ph)
          process(a + ph, ph)
        return carry

      lax.fori_loop(0, nchunks // NBUF, outer, 0)

    @pl.when(c == 0)
    def _():
      run(fake_hbm)

    @pl.when(c == 1)
    def _():
      run(real_hbm)

    for j in range(TBL // L):
      v = zeros
      for d in range(L):
        diag = (jnp.arange(L, dtype=jnp.int32) * L + j * L * L +
                (jnp.arange(L, dtype=jnp.int32) + d) % L)
        v = v + plsc.load_gather(table, [diag])
      acc[pl.ds(j * L, L)] = v

    pltpu.sync_copy(acc, out_hbm.at[c * NS + s])

  return hist_kernel(fake_images, real_images)


def _loss_body(h_ref, o_ref):
  h = h_ref[...]
  ssum = jnp.clip(jnp.sum(h, axis=1, keepdims=True), 1e-8, None)
  n = h / ssum
  d = jnp.abs(n[:ROWS] - n[ROWS:])
  o_ref[0, 0] = jnp.sum(d) / jnp.float32(ROWS * BINS)


def kernel(fake_images, real_images):
  hists = _histograms_sc(fake_images, real_images).reshape(NC * ROWS, BINS)
  loss = pl.pallas_call(
      _loss_body,
      out_shape=jax.ShapeDtypeStruct((1, 1), jnp.float32),
      out_specs=pl.BlockSpec(memory_space=pltpu.SMEM),
  )(hists)
  return loss[0, 0]


# speedup vs baseline: 1.1782x; 1.0079x over previous
import functools

import jax
import jax.numpy as jnp
from jax import lax
from jax.experimental import pallas as pl
from jax.experimental.pallas import tpu as pltpu
from jax.experimental.pallas import tpu_sc as plsc

B, C, H, W = 16, 3, 512, 512
BINS = 64
ROWS = B * C

NC, NS, L = 2, 16, 16
CH_ROWS = 32
CHUNK = CH_ROWS * W
CHUNKS_PER_CH = H // CH_ROWS
CH_SHIFT = 4
NBUF = 4
VECS_PER_ROW = W // L
TBL = C * BINS


def _histograms_sc(fake_images, real_images):
  mesh = plsc.VectorSubcoreMesh(
      core_axis_name="c", subcore_axis_name="s", num_cores=NC,
      num_subcores=NS)

  @functools.partial(
      pl.kernel,
      out_type=jax.ShapeDtypeStruct((NC * NS, TBL), jnp.float32),
      mesh=mesh,
      compiler_params=pltpu.CompilerParams(
          needs_layout_passes=False, use_tc_tiling_on_sc=True),
      scratch_types=[
          [pltpu.VMEM((CH_ROWS, W), jnp.float32)] * NBUF,
          pltpu.VMEM((L * TBL,), jnp.float32),
          pltpu.VMEM((TBL,), jnp.float32),
          [pltpu.SemaphoreType.DMA] * NBUF,
      ],
  )
  def hist_kernel(fake_hbm, real_hbm, out_hbm, bufs, table, acc, sems):
    c = lax.axis_index("c")
    s = lax.axis_index("s")
    lane_iota = jnp.arange(L, dtype=jnp.int32)
    ones = jnp.ones((L,), jnp.float32)
    zeros = jnp.zeros((L,), jnp.float32)
    nchunks = C * CHUNKS_PER_CH

    @plsc.parallel_loop(0, TBL, unroll=8)
    def _(j):
      table[pl.ds(j * L, L)] = zeros

    def run(img_hbm):

      def start(k, nb):
        ch = lax.shift_right_logical(k, CH_SHIFT)
        blk = lax.bitwise_and(k, CHUNKS_PER_CH - 1)
        pltpu.async_copy(
            img_hbm.at[s, ch, pl.ds(blk * CH_ROWS, CH_ROWS), :],
            bufs[nb], sems[nb])

      def wait(nb):
        pltpu.make_async_copy(
            img_hbm.at[0, 0, pl.ds(0, CH_ROWS), :], bufs[nb],
            sems[nb]).wait()

      def process(k, nb):
        cvec = lane_iota + lax.shift_right_logical(k, CH_SHIFT) * (BINS * L)

        @plsc.parallel_loop(0, CH_ROWS)
        def _(r):

          @plsc.parallel_loop(0, VECS_PER_ROW, unroll=8)
          def _(col):
            x = bufs[nb][r, pl.ds(col * L, L)]
            idx = (x * jnp.float32(BINS)).astype(jnp.int32)
            idx = plsc.bitcast(
                jnp.minimum(plsc.bitcast(idx, jnp.uint32),
                            jnp.uint32(BINS - 1)), jnp.int32)
            addr = lax.shift_left(idx, 4) + cvec
            plsc.addupdate_scatter(table, [addr], ones)

      for i in range(NBUF - 1):
        start(jnp.int32(i), i)

      def outer(kq, carry):
        a = kq * NBUF
        for ph in range(NBUF):

          @pl.when(a + ph + NBUF - 1 < nchunks)
          def _(ph=ph):
            start(a + ph + NBUF - 1, (ph + NBUF - 1) % NBUF)

          wait(ph)
          process(a + ph, ph)
        return carry

      lax.fori_loop(0, nchunks // NBUF, outer, 0)

    @pl.when(c == 0)
    def _():
      run(fake_hbm)

    @pl.when(c == 1)
    def _():
      run(real_hbm)

    for j in range(TBL // L):
      v = zeros
      for d in range(L):
        diag = (jnp.arange(L, dtype=jnp.int32) * L + j * L * L +
                (jnp.arange(L, dtype=jnp.int32) + d) % L)
        v = v + plsc.load_gather(table, [diag])
      acc[pl.ds(j * L, L)] = v

    pltpu.sync_copy(acc, out_hbm.at[c * NS + s])

  return hist_kernel(fake_images, real_images)


def _loss_body(h_ref, o_ref):
  h = h_ref[...]
  ssum = jnp.clip(jnp.sum(h, axis=1, keepdims=True), 1e-8, None)
  n = h / ssum
  d = jnp.abs(n[:ROWS] - n[ROWS:])
  o_ref[0, 0] = jnp.sum(d) / jnp.float32(ROWS * BINS)


def kernel(fake_images, real_images):
  hists = _histograms_sc(fake_images, real_images).reshape(NC * ROWS, BINS)
  loss = pl.pallas_call(
      _loss_body,
      out_shape=jax.ShapeDtypeStruct((1, 1), jnp.float32),
      out_specs=pl.BlockSpec(memory_space=pltpu.SMEM),
  )(hists)
  return loss[0, 0]


# speedup vs baseline: 1.4629x; 1.2416x over previous
import functools

import jax
import jax.numpy as jnp
from jax import lax
from jax.experimental import pallas as pl
from jax.experimental.pallas import tpu as pltpu
from jax.experimental.pallas import tpu_sc as plsc

B, C, H, W = 16, 3, 512, 512
BINS = 64
ROWS = B * C

NC, NS, L = 2, 16, 16
CH_ROWS = 32
CHUNK = CH_ROWS * W
CHUNKS_PER_CH = H // CH_ROWS
CH_SHIFT = 4
NBUF = 4
VECS_PER_ROW = W // L
TBL = C * BINS


def _histograms_sc(fake_images, real_images):
  mesh = plsc.VectorSubcoreMesh(
      core_axis_name="c", subcore_axis_name="s", num_cores=NC,
      num_subcores=NS)

  @functools.partial(
      pl.kernel,
      out_type=jax.ShapeDtypeStruct((NC * NS, TBL), jnp.float32),
      mesh=mesh,
      compiler_params=pltpu.CompilerParams(
          needs_layout_passes=False, use_tc_tiling_on_sc=True),
      scratch_types=[
          [pltpu.VMEM((CH_ROWS, W), jnp.float32)] * NBUF,
          pltpu.VMEM((L * TBL,), jnp.float32),
          pltpu.VMEM((TBL,), jnp.float32),
          [pltpu.SemaphoreType.DMA] * NBUF,
      ],
  )
  def hist_kernel(fake_hbm, real_hbm, out_hbm, bufs, table, acc, sems):
    c = lax.axis_index("c")
    s = lax.axis_index("s")
    lane_iota = jnp.arange(L, dtype=jnp.int32)
    ones = jnp.ones((L,), jnp.float32)
    zeros = jnp.zeros((L,), jnp.float32)
    nchunks = C * CHUNKS_PER_CH

    @plsc.parallel_loop(0, TBL, unroll=8)
    def _(j):
      table[pl.ds(j * L, L)] = zeros

    def run(img_hbm):

      def start(k, nb):
        ch = lax.shift_right_logical(k, CH_SHIFT)
        blk = lax.bitwise_and(k, CHUNKS_PER_CH - 1)
        pltpu.async_copy(
            img_hbm.at[s, ch, pl.ds(blk * CH_ROWS, CH_ROWS), :],
            bufs[nb], sems[nb])

      def wait(nb):
        pltpu.make_async_copy(
            img_hbm.at[0, 0, pl.ds(0, CH_ROWS), :], bufs[nb],
            sems[nb]).wait()

      def process(k, nb):
        cvec = lane_iota + lax.shift_right_logical(k, CH_SHIFT) * (BINS * L)

        @plsc.parallel_loop(0, CHUNK // L, unroll=8)
        def _(i):
            r = lax.shift_right_logical(i, 5)
            cw = lax.shift_left(lax.bitwise_and(i, VECS_PER_ROW - 1), 4)
            x = bufs[nb][r, pl.ds(cw, L)]
            idx = (x * jnp.float32(BINS)).astype(jnp.int32)
            idx = plsc.bitcast(
                jnp.minimum(plsc.bitcast(idx, jnp.uint32),
                            jnp.uint32(BINS - 1)), jnp.int32)
            addr = lax.shift_left(idx, 4) + cvec
            plsc.addupdate_scatter(table, [addr], ones)

      for i in range(NBUF - 1):
        start(jnp.int32(i), i)

      def outer(kq, carry):
        a = kq * NBUF
        for ph in range(NBUF):

          @pl.when(a + ph + NBUF - 1 < nchunks)
          def _(ph=ph):
            start(a + ph + NBUF - 1, (ph + NBUF - 1) % NBUF)

          wait(ph)
          process(a + ph, ph)
        return carry

      lax.fori_loop(0, nchunks // NBUF, outer, 0)

    @pl.when(c == 0)
    def _():
      run(fake_hbm)

    @pl.when(c == 1)
    def _():
      run(real_hbm)

    for j in range(TBL // L):
      v = zeros
      for d in range(L):
        diag = (jnp.arange(L, dtype=jnp.int32) * L + j * L * L +
                (jnp.arange(L, dtype=jnp.int32) + d) % L)
        v = v + plsc.load_gather(table, [diag])
      acc[pl.ds(j * L, L)] = v

    pltpu.sync_copy(acc, out_hbm.at[c * NS + s])

  return hist_kernel(fake_images, real_images)


def _loss_body(h_ref, o_ref):
  h = h_ref[...]
  ssum = jnp.clip(jnp.sum(h, axis=1, keepdims=True), 1e-8, None)
  n = h / ssum
  d = jnp.abs(n[:ROWS] - n[ROWS:])
  o_ref[0, 0] = jnp.sum(d) / jnp.float32(ROWS * BINS)


def kernel(fake_images, real_images):
  hists = _histograms_sc(fake_images, real_images).reshape(NC * ROWS, BINS)
  loss = pl.pallas_call(
      _loss_body,
      out_shape=jax.ShapeDtypeStruct((1, 1), jnp.float32),
      out_specs=pl.BlockSpec(memory_space=pltpu.SMEM),
  )(hists)
  return loss[0, 0]
